# baseline (speedup 1.0000x reference)
import jax
import jax.numpy as jnp
from jax import lax
from jax.experimental import pallas as pl
from jax.experimental.pallas import tpu as pltpu
from jax.experimental.pallas import tpu_sc as plsc

BB = 4096
LL = 50
DD = 64
N = BB * LL
VOCAB = 1000000

NC, NS = 2, 16
NW = NC * NS
LP = 56
NP = BB * LP
PER_W = NP // NW
CHUNK = 224
NCHUNK = PER_W // CHUNK
NBUF = 2

CMM = 16384


def _mm_body(t_ref, wd_ref, bd_ref, o_ref):
    o_ref[...] = (
        lax.dot_general(
            t_ref[...],
            wd_ref[...],
            (((0,), (0,)), ((), ())),
            preferred_element_type=jnp.float32,
        )
        + bd_ref[...]
    )


_mm = pl.pallas_call(
    _mm_body,
    grid=(pl.cdiv(VOCAB, CMM),),
    in_specs=[
        pl.BlockSpec((DD, CMM), lambda i: (0, i)),
        pl.BlockSpec((DD, 2 * DD), lambda i: (0, 0)),
        pl.BlockSpec((1, 2 * DD), lambda i: (0, 0)),
    ],
    out_specs=pl.BlockSpec((CMM, 2 * DD), lambda i: (i, 0)),
    out_shape=jax.ShapeDtypeStruct((VOCAB, 2 * DD), jnp.float32),
    compiler_params=pltpu.CompilerParams(fuse_transposed_lhs_in_matmul=True),
)


def _sc_gather_body(
    idx_hbm, cache_hbm, out_hbm, idx_v, rows0, rows1, sg0, sg1, ss0, ss1
):
    wid = lax.axis_index("s") * NC + lax.axis_index("c")
    base = wid * PER_W
    pltpu.sync_copy(idx_hbm.at[pl.ds(base, PER_W)], idx_v)
    rows, sg, ss = (rows0, rows1), (sg0, sg1), (ss0, ss1)

    def gather_start(j, slot):
        pltpu.async_copy(
            cache_hbm.at[idx_v.at[pl.ds(j * CHUNK, CHUNK)]], rows[slot], sg[slot]
        )

    def gather_wait(slot):
        pltpu.make_async_copy(
            cache_hbm.at[idx_v.at[pl.ds(0, CHUNK)]], rows[slot], sg[slot]
        ).wait()

    def store_start(j, slot):
        pltpu.async_copy(
            rows[slot], out_hbm.at[pl.ds(base + j * CHUNK, CHUNK)], ss[slot]
        )

    def store_wait(slot):
        pltpu.make_async_copy(
            rows[slot], out_hbm.at[pl.ds(base, CHUNK)], ss[slot]
        ).wait()

    def step(t, carry):
        for slot in range(NBUF):
            j = t * NBUF + slot
            other = 1 - slot

            @pl.when(j >= 1)
            def _():
                gather_wait(other)
                store_start(j - 1, other)

            @pl.when(j >= NBUF)
            def _():
                store_wait(slot)

            gather_start(j, slot)
        return carry

    lax.fori_loop(0, NCHUNK // NBUF, step, 0)
    last = NCHUNK - 1
    gather_wait(last % NBUF)
    store_start(last, last % NBUF)
    store_wait((last - 1) % NBUF)
    store_wait(last % NBUF)


_sc_gather = pl.kernel(
    _sc_gather_body,
    out_type=jax.ShapeDtypeStruct((NP, 2 * DD), jnp.float32),
    mesh=plsc.VectorSubcoreMesh(
        core_axis_name="c", subcore_axis_name="s", num_cores=NC, num_subcores=NS
    ),
    scratch_types=[
        pltpu.VMEM((PER_W,), jnp.int32),
        pltpu.VMEM((CHUNK, 2 * DD), jnp.float32),
        pltpu.VMEM((CHUNK, 2 * DD), jnp.float32),
        pltpu.SemaphoreType.DMA,
        pltpu.SemaphoreType.DMA,
        pltpu.SemaphoreType.DMA,
        pltpu.SemaphoreType.DMA,
    ],
    compiler_params=pltpu.CompilerParams(use_tc_tiling_on_sc=True),
)


@jax.jit
def kernel(x, table, W, b):
    xi = x.astype(jnp.int32)
    xp = jnp.concatenate([xi, xi[:, : LP - LL]], axis=1)
    xf = xp.reshape(-1)
    wt = W.T
    wd = jnp.concatenate([wt, wt], axis=1)
    bd = jnp.concatenate([b, b]).reshape(1, 2 * DD)
    cache2 = _mm(table.T, wd, bd)
    g2 = _sc_gather(xf, cache2)
    return g2.reshape(BB, LP, 2 * DD)[:, :LL, :DD]


# speedup vs baseline: 1.0148x; 1.0148x over previous
import jax
import jax.numpy as jnp
from jax import lax
from jax.experimental import pallas as pl
from jax.experimental.pallas import tpu as pltpu
from jax.experimental.pallas import tpu_sc as plsc

BB = 4096
LL = 50
DD = 64
N = BB * LL
VOCAB = 1000000

NC, NS = 2, 16
NW = NC * NS
LP = 56
NP = BB * LP
PER_W = NP // NW
CHUNK = 448
NCHUNK = PER_W // CHUNK
NBUF = 2

CMM = 24576


def _mm_body(t_ref, wd_ref, bd_ref, o_ref):
    o_ref[...] = (
        lax.dot_general(
            t_ref[...],
            wd_ref[...],
            (((0,), (0,)), ((), ())),
            preferred_element_type=jnp.float32,
        )
        + bd_ref[...]
    )


_mm = pl.pallas_call(
    _mm_body,
    grid=(pl.cdiv(VOCAB, CMM),),
    in_specs=[
        pl.BlockSpec((DD, CMM), lambda i: (0, i)),
        pl.BlockSpec((DD, 2 * DD), lambda i: (0, 0)),
        pl.BlockSpec((1, 2 * DD), lambda i: (0, 0)),
    ],
    out_specs=pl.BlockSpec((CMM, 2 * DD), lambda i: (i, 0)),
    out_shape=jax.ShapeDtypeStruct((VOCAB, 2 * DD), jnp.float32),
    compiler_params=pltpu.CompilerParams(fuse_transposed_lhs_in_matmul=True),
)


def _sc_gather_body(
    idx_hbm, cache_hbm, out_hbm, idx_v, rows0, rows1, sg0, sg1, ss0, ss1
):
    wid = lax.axis_index("s") * NC + lax.axis_index("c")
    base = wid * PER_W
    pltpu.sync_copy(idx_hbm.at[pl.ds(base, PER_W)], idx_v)
    rows, sg, ss = (rows0, rows1), (sg0, sg1), (ss0, ss1)

    def gather_start(j, slot):
        pltpu.async_copy(
            cache_hbm.at[idx_v.at[pl.ds(j * CHUNK, CHUNK)]], rows[slot], sg[slot]
        )

    def gather_wait(slot):
        pltpu.make_async_copy(
            cache_hbm.at[idx_v.at[pl.ds(0, CHUNK)]], rows[slot], sg[slot]
        ).wait()

    def store_start(j, slot):
        pltpu.async_copy(
            rows[slot], out_hbm.at[pl.ds(base + j * CHUNK, CHUNK)], ss[slot]
        )

    def store_wait(slot):
        pltpu.make_async_copy(
            rows[slot], out_hbm.at[pl.ds(base, CHUNK)], ss[slot]
        ).wait()

    def step(t, carry):
        for slot in range(NBUF):
            j = t * NBUF + slot
            other = 1 - slot

            @pl.when(j >= 1)
            def _():
                gather_wait(other)
                store_start(j - 1, other)

            @pl.when(j >= NBUF)
            def _():
                store_wait(slot)

            gather_start(j, slot)
        return carry

    lax.fori_loop(0, NCHUNK // NBUF, step, 0)
    last = NCHUNK - 1
    gather_wait(last % NBUF)
    store_start(last, last % NBUF)
    store_wait((last - 1) % NBUF)
    store_wait(last % NBUF)


_sc_gather = pl.kernel(
    _sc_gather_body,
    out_type=jax.ShapeDtypeStruct((NP, 2 * DD), jnp.float32),
    mesh=plsc.VectorSubcoreMesh(
        core_axis_name="c", subcore_axis_name="s", num_cores=NC, num_subcores=NS
    ),
    scratch_types=[
        pltpu.VMEM((PER_W,), jnp.int32),
        pltpu.VMEM((CHUNK, 2 * DD), jnp.float32),
        pltpu.VMEM((CHUNK, 2 * DD), jnp.float32),
        pltpu.SemaphoreType.DMA,
        pltpu.SemaphoreType.DMA,
        pltpu.SemaphoreType.DMA,
        pltpu.SemaphoreType.DMA,
    ],
    compiler_params=pltpu.CompilerParams(use_tc_tiling_on_sc=True),
)


@jax.jit
def kernel(x, table, W, b):
    xi = x.astype(jnp.int32)
    xp = jnp.concatenate([xi, xi[:, : LP - LL]], axis=1)
    xf = xp.reshape(-1)
    wt = W.T
    wd = jnp.concatenate([wt, wt], axis=1)
    bd = jnp.concatenate([b, b]).reshape(1, 2 * DD)
    cache2 = _mm(table.T, wd, bd)
    g2 = _sc_gather(xf, cache2)
    return g2.reshape(BB, LP, 2 * DD)[:, :LL, :DD]
